# Initial kernel scaffold; baseline (speedup 1.0000x reference)
import jax
import jax.numpy as jnp
from jax.experimental import pallas as pl


def kernel(x, edge_index, edge_mask, W, b, gamma, beta, prelu_a):
    raise NotImplementedError("write your pallas kernel here")


# speedup vs baseline: 13.5113x; 13.5113x over previous
import functools

import jax
import jax.numpy as jnp
from jax import lax
from jax.experimental import pallas as pl
from jax.experimental.pallas import tpu as pltpu
from jax.experimental.pallas import tpu_sc as plsc

B = 4
N = 10000
K = 17
D = 128
NN = B * N
EB = N * 16
TRASH = N
ACC_ROWS = N + 16
NT = 16
NODES_T = 640
CHUNK_E = 128
NCHUNKS = EB // CHUNK_E
GROWS = 80


def _matmul_body(x_ref, w_ref, h_ref):
    h_ref[...] = jnp.dot(x_ref[...], w_ref[...],
                         preferred_element_type=jnp.float32)


def _edge_prep_body(ei_ref, em_ref, dst_ref, src_ref):
    step = pl.program_id(0)
    ei = ei_ref[...][:, :, 1:]
    em = em_ref[...][:, :, 1:]
    dst_ref[...] = jnp.where(em != 0, ei, TRASH)
    bidx = lax.broadcasted_iota(jnp.int32, (B, 1000, 16), 0)
    iidx = lax.broadcasted_iota(jnp.int32, (B, 1000, 16), 1) + step * 1000
    src_ref[...] = bidx * N + iidx


def _sc_body(h_hbm, dst_hbm, srcrep_hbm,
             acc_hbm, dinv_hbm, g_hbm,
             spmem_acc, spmem_deg,
             ones_v, idx_v, msg_v, row_v, dnv_v, sem):
    c = lax.axis_index("c")
    t = lax.axis_index("s")
    last = t == NT - 1
    start = t * NODES_T

    def _fill_ones(i, _):
        ones_v[pl.ds(i * 16, 16)] = jnp.full((16,), 1.0, jnp.float32)
        return 0
    lax.fori_loop(0, NODES_T // 16, _fill_ones, 0)

    for lb in range(2):
        b = c * 2 + lb
        ebase = b * EB

        @pl.when(~last)
        def _():
            pltpu.sync_copy(ones_v.at[pl.ds(0, NODES_T)],
                            spmem_deg.at[pl.ds(start, NODES_T)])

        @pl.when(last)
        def _():
            pltpu.sync_copy(ones_v.at[pl.ds(0, 416)],
                            spmem_deg.at[pl.ds(start, 416)])

        plsc.subcore_barrier()

        def _deg_chunk(i, _):
            cid = t + i * NT
            @pl.when(cid < NCHUNKS)
            def _():
                pltpu.sync_copy(
                    dst_hbm.at[pl.ds(ebase + cid * CHUNK_E, CHUNK_E)],
                    idx_v)
                pltpu.sync_copy(ones_v.at[pl.ds(0, CHUNK_E)],
                                spmem_deg.at[idx_v], add=True)
            return 0
        lax.fori_loop(0, (NCHUNKS + NT - 1) // NT, _deg_chunk, 0)

        plsc.subcore_barrier()

        @pl.when(~last)
        def _():
            pltpu.sync_copy(spmem_deg.at[pl.ds(start, NODES_T)],
                            dnv_v.at[pl.ds(0, NODES_T)])

        @pl.when(last)
        def _():
            pltpu.sync_copy(spmem_deg.at[pl.ds(start, 400)],
                            dnv_v.at[pl.ds(0, 400)])

        cnt_nodes = jnp.where(last, 400, NODES_T)

        def _newton(i, _):
            @pl.when(i * 16 < cnt_nodes)
            def _():
                x = dnv_v[pl.ds(i * 16, 16)]
                bits = lax.bitcast_convert_type(x, jnp.int32)
                seed = jnp.full((16,), 0x5F3759DF, jnp.int32) - (
                    lax.shift_right_logical(bits, 1))
                y = lax.bitcast_convert_type(seed, jnp.float32)
                for _it in range(4):
                    y = y * (1.5 - 0.5 * x * y * y)
                dnv_v[pl.ds(i * 16, 16)] = y
            return 0
        lax.fori_loop(0, NODES_T // 16, _newton, 0)

        @pl.when(~last)
        def _():
            pltpu.sync_copy(dnv_v.at[pl.ds(0, NODES_T)],
                            dinv_hbm.at[pl.ds(b * N + start, NODES_T)])

        @pl.when(last)
        def _():
            pltpu.sync_copy(dnv_v.at[pl.ds(0, 400)],
                            dinv_hbm.at[pl.ds(b * N + start, 400)])

        def _g_chunk(ci, _):
            r0 = ci * GROWS
            @pl.when(r0 < cnt_nodes)
            def _():
                gbase = b * N + start + r0
                pltpu.sync_copy(h_hbm.at[pl.ds(gbase, GROWS)], row_v)

                def _scale16(rr, _2):
                    dvec = dnv_v[pl.ds(r0 + rr * 16, 16)]
                    for l in range(16):
                        dv = jnp.full((16,), 1.0, jnp.float32) * dvec[l]
                        r = rr * 16 + l
                        for j in range(D // 16):
                            row_v[r, pl.ds(j * 16, 16)] = (
                                row_v[r, pl.ds(j * 16, 16)] * dv)
                    return 0
                lax.fori_loop(0, GROWS // 16, _scale16, 0)
                pltpu.sync_copy(row_v, g_hbm.at[pl.ds(gbase, GROWS)])
            return 0
        lax.fori_loop(0, NODES_T // GROWS, _g_chunk, 0)

    plsc.subcore_barrier()

    for lb in range(2):
        b = c * 2 + lb
        ebase = b * EB

        @pl.when(~last)
        def _():
            pltpu.sync_copy(g_hbm.at[pl.ds(b * N + start, NODES_T)],
                            spmem_acc.at[pl.ds(start, NODES_T)])

        @pl.when(last)
        def _():
            pltpu.sync_copy(g_hbm.at[pl.ds(b * N + start, 400)],
                            spmem_acc.at[pl.ds(start, 400)])
            def _zt(i, _):
                msg_v[0, pl.ds(i * 16, 16)] = jnp.zeros((16,), jnp.float32)
                return 0
            lax.fori_loop(0, D // 16, _zt, 0)
            for tr in range(16):
                pltpu.sync_copy(msg_v.at[0], spmem_acc.at[N + tr])

        plsc.subcore_barrier()

        def _msg_chunk(i, _):
            cid = t + i * NT
            @pl.when(cid < NCHUNKS)
            def _():
                e0 = ebase + cid * CHUNK_E
                pltpu.sync_copy(srcrep_hbm.at[pl.ds(e0, CHUNK_E)], idx_v)
                pltpu.async_copy(g_hbm.at[idx_v], msg_v, sem).wait()
                pltpu.sync_copy(dst_hbm.at[pl.ds(e0, CHUNK_E)], idx_v)
                pltpu.sync_copy(msg_v, spmem_acc.at[idx_v], add=True)
            return 0
        lax.fori_loop(0, (NCHUNKS + NT - 1) // NT, _msg_chunk, 0)

        plsc.subcore_barrier()

        @pl.when(~last)
        def _():
            pltpu.sync_copy(spmem_acc.at[pl.ds(start, NODES_T)],
                            acc_hbm.at[pl.ds(b * N + start, NODES_T)])

        @pl.when(last)
        def _():
            pltpu.sync_copy(spmem_acc.at[pl.ds(start, 400)],
                            acc_hbm.at[pl.ds(b * N + start, 400)])

        plsc.subcore_barrier()


def _stats_body(acc_ref, dinv_ref, bias_ref, stats_ref, accum):
    step = pl.program_id(0)

    @pl.when(step == 0)
    def _():
        accum[0] = 0.0
        accum[1] = 0.0

    y = dinv_ref[...] * acc_ref[...] + bias_ref[...]
    accum[0] += jnp.sum(y)
    accum[1] += jnp.sum(y * y)

    @pl.when(step == pl.num_programs(0) - 1)
    def _():
        stats_ref[0] = accum[0]
        stats_ref[1] = accum[1]


def _final_body(acc_ref, dinv_ref, bias_ref, stats_ref,
                gamma_ref, beta_ref, a_ref, out_ref):
    mu = stats_ref[0] / (NN * D)
    var = stats_ref[1] / (NN * D) - mu * mu
    rs = lax.rsqrt(var + 1e-5)
    y = dinv_ref[...] * acc_ref[...] + bias_ref[...]
    y = (y - mu) * rs * gamma_ref[...] + beta_ref[...]
    out_ref[...] = jnp.where(y >= 0, y, a_ref[...] * y)


def kernel(x, edge_index, edge_mask, W, b, gamma, beta, prelu_a):
    x2 = x[:, 0, :]
    ei32 = edge_index.astype(jnp.int32)
    em32 = edge_mask.astype(jnp.int32)

    h = pl.pallas_call(
        _matmul_body,
        grid=(20,),
        in_specs=[pl.BlockSpec((2000, D), lambda i: (i, 0)),
                  pl.BlockSpec((D, D), lambda i: (0, 0))],
        out_specs=pl.BlockSpec((2000, D), lambda i: (i, 0)),
        out_shape=jax.ShapeDtypeStruct((NN, D), jnp.float32),
    )(x2, W)

    dst_pad, src_rep = pl.pallas_call(
        _edge_prep_body,
        grid=(10,),
        in_specs=[pl.BlockSpec((B, 1000, K), lambda i: (0, i, 0)),
                  pl.BlockSpec((B, 1000, K), lambda i: (0, i, 0))],
        out_specs=[pl.BlockSpec((B, 1000, 16), lambda i: (0, i, 0)),
                   pl.BlockSpec((B, 1000, 16), lambda i: (0, i, 0))],
        out_shape=[jax.ShapeDtypeStruct((B, N, 16), jnp.int32),
                   jax.ShapeDtypeStruct((B, N, 16), jnp.int32)],
    )(ei32, em32)
    dst_flat = dst_pad.reshape(-1)
    src_flat = src_rep.reshape(-1)

    mesh = plsc.VectorSubcoreMesh(core_axis_name="c", subcore_axis_name="s")
    acc, dinv, g = pl.kernel(
        _sc_body,
        out_type=(jax.ShapeDtypeStruct((NN, D), jnp.float32),
                  jax.ShapeDtypeStruct((NN,), jnp.float32),
                  jax.ShapeDtypeStruct((NN, D), jnp.float32)),
        mesh=mesh,
        scratch_types=[
            pltpu.VMEM_SHARED((ACC_ROWS, D), jnp.float32),
            pltpu.VMEM_SHARED((ACC_ROWS,), jnp.float32),
            pltpu.VMEM((NODES_T,), jnp.float32),
            pltpu.VMEM((CHUNK_E,), jnp.int32),
            pltpu.VMEM((CHUNK_E, D), jnp.float32),
            pltpu.VMEM((GROWS, D), jnp.float32),
            pltpu.VMEM((NODES_T,), jnp.float32),
            pltpu.SemaphoreType.DMA,
        ],
    )(h, dst_flat, src_flat)

    dinv2 = dinv.reshape(NN, 1)
    bias2 = b.reshape(1, D)
    stats = pl.pallas_call(
        _stats_body,
        grid=(20,),
        in_specs=[pl.BlockSpec((2000, D), lambda i: (i, 0)),
                  pl.BlockSpec((2000, 1), lambda i: (i, 0)),
                  pl.BlockSpec((1, D), lambda i: (0, 0))],
        out_specs=pl.BlockSpec(memory_space=pltpu.SMEM),
        out_shape=jax.ShapeDtypeStruct((2,), jnp.float32),
        scratch_shapes=[pltpu.SMEM((2,), jnp.float32)],
    )(acc, dinv2, bias2)

    out = pl.pallas_call(
        _final_body,
        grid=(20,),
        in_specs=[pl.BlockSpec((2000, D), lambda i: (i, 0)),
                  pl.BlockSpec((2000, 1), lambda i: (i, 0)),
                  pl.BlockSpec((1, D), lambda i: (0, 0)),
                  pl.BlockSpec(memory_space=pltpu.SMEM),
                  pl.BlockSpec((1, D), lambda i: (0, 0)),
                  pl.BlockSpec((1, D), lambda i: (0, 0)),
                  pl.BlockSpec((1, 1), lambda i: (0, 0))],
        out_specs=pl.BlockSpec((2000, D), lambda i: (i, 0)),
        out_shape=jax.ShapeDtypeStruct((NN, D), jnp.float32),
    )(acc, dinv2, bias2, stats, gamma.reshape(1, D), beta.reshape(1, D),
      prelu_a.reshape(1, 1))
    return out


# speedup vs baseline: 14.6478x; 1.0841x over previous
import functools

import jax
import jax.numpy as jnp
from jax import lax
from jax.experimental import pallas as pl
from jax.experimental.pallas import tpu as pltpu
from jax.experimental.pallas import tpu_sc as plsc

B = 4
N = 10000
K = 17
D = 128
NN = B * N
EB = N * 16
TRASH = N
ACC_ROWS = N + 16
NT = 16
NODES_T = 640
CHUNK_E = 128
NCHUNKS = EB // CHUNK_E
GROWS = 80


def _matmul_body(x_ref, w_ref, h_ref):
    h_ref[...] = jnp.dot(x_ref[...], w_ref[...],
                         preferred_element_type=jnp.float32)


def _edge_prep_body(ei_ref, em_ref, dst_ref, src_ref):
    step = pl.program_id(0)
    ei = ei_ref[...][:, :, 1:]
    em = em_ref[...][:, :, 1:]
    dst_ref[...] = jnp.where(em != 0, ei, TRASH)
    bidx = lax.broadcasted_iota(jnp.int32, (B, 1000, 16), 0)
    iidx = lax.broadcasted_iota(jnp.int32, (B, 1000, 16), 1) + step * 1000
    src_ref[...] = bidx * N + iidx


def _sc_body(h_hbm, dst_hbm, srcrep_hbm,
             acc_hbm, dinv_hbm, g_hbm,
             spmem_acc, spmem_degf,
             one_v, zer_v, idx_v, msg_v, row_v, dnv_v, red_v, sem):
    c = lax.axis_index("c")
    t = lax.axis_index("s")
    last = t == NT - 1
    start = t * NODES_T
    dbase = t * ACC_ROWS

    def _fill_one(i, _):
        one_v[pl.ds(i * 16, 16)] = jnp.full((16,), 1.0, jnp.float32)
        return 0
    lax.fori_loop(0, CHUNK_E // 16, _fill_one, 0)

    def _fill_zer(i, _):
        zer_v[pl.ds(i * 16, 16)] = jnp.zeros((16,), jnp.float32)
        return 0
    lax.fori_loop(0, NODES_T // 16, _fill_zer, 0)

    for lb in range(2):
        b = c * 2 + lb
        ebase = b * EB

        def _zero_own(j, _):
            pltpu.sync_copy(zer_v,
                            spmem_degf.at[pl.ds(dbase + j * NODES_T,
                                                NODES_T)])
            return 0
        lax.fori_loop(0, 15, _zero_own, 0)
        pltpu.sync_copy(zer_v.at[pl.ds(0, ACC_ROWS - 15 * NODES_T)],
                        spmem_degf.at[pl.ds(dbase + 15 * NODES_T,
                                            ACC_ROWS - 15 * NODES_T)])

        def _deg_chunk(i, _):
            cid = t + i * NT
            @pl.when(cid < NCHUNKS)
            def _():
                pltpu.sync_copy(
                    dst_hbm.at[pl.ds(ebase + cid * CHUNK_E, CHUNK_E)],
                    idx_v)
                for j in range(CHUNK_E // 16):
                    idx_v[pl.ds(j * 16, 16)] = (
                        idx_v[pl.ds(j * 16, 16)] + dbase)
                pltpu.sync_copy(one_v, spmem_degf.at[idx_v], add=True)
            return 0
        lax.fori_loop(0, (NCHUNKS + NT - 1) // NT, _deg_chunk, 0)

        plsc.subcore_barrier()

        for k in range(NT):
            pltpu.sync_copy(
                spmem_degf.at[pl.ds(k * ACC_ROWS + start, NODES_T)],
                red_v.at[pl.ds(k * NODES_T, NODES_T)])

        cnt_nodes = jnp.where(last, 400, NODES_T)

        def _newton(i, _):
            @pl.when(i * 16 < cnt_nodes)
            def _():
                x = jnp.full((16,), 1.0, jnp.float32)
                for k in range(NT):
                    x = x + red_v[pl.ds(k * NODES_T + i * 16, 16)]
                bits = lax.bitcast_convert_type(x, jnp.int32)
                seed = jnp.full((16,), 0x5F3759DF, jnp.int32) - (
                    lax.shift_right_logical(bits, 1))
                y = lax.bitcast_convert_type(seed, jnp.float32)
                for _it in range(4):
                    y = y * (1.5 - 0.5 * x * y * y)
                dnv_v[pl.ds(i * 16, 16)] = y
            return 0
        lax.fori_loop(0, NODES_T // 16, _newton, 0)

        plsc.subcore_barrier()

        @pl.when(~last)
        def _():
            pltpu.sync_copy(dnv_v.at[pl.ds(0, NODES_T)],
                            dinv_hbm.at[pl.ds(b * N + start, NODES_T)])

        @pl.when(last)
        def _():
            pltpu.sync_copy(dnv_v.at[pl.ds(0, 400)],
                            dinv_hbm.at[pl.ds(b * N + start, 400)])

        def _g_chunk(ci, _):
            r0 = ci * GROWS
            @pl.when(r0 < cnt_nodes)
            def _():
                gbase = b * N + start + r0
                pltpu.sync_copy(h_hbm.at[pl.ds(gbase, GROWS)], row_v)

                def _scale16(rr, _2):
                    dvec = dnv_v[pl.ds(r0 + rr * 16, 16)]
                    for l in range(16):
                        dv = jnp.full((16,), 1.0, jnp.float32) * dvec[l]
                        r = rr * 16 + l
                        for j in range(D // 16):
                            row_v[r, pl.ds(j * 16, 16)] = (
                                row_v[r, pl.ds(j * 16, 16)] * dv)
                    return 0
                lax.fori_loop(0, GROWS // 16, _scale16, 0)
                pltpu.sync_copy(row_v, g_hbm.at[pl.ds(gbase, GROWS)])
            return 0
        lax.fori_loop(0, NODES_T // GROWS, _g_chunk, 0)

    plsc.subcore_barrier()

    for lb in range(2):
        b = c * 2 + lb
        ebase = b * EB

        @pl.when(~last)
        def _():
            pltpu.sync_copy(g_hbm.at[pl.ds(b * N + start, NODES_T)],
                            spmem_acc.at[pl.ds(start, NODES_T)])

        @pl.when(last)
        def _():
            pltpu.sync_copy(g_hbm.at[pl.ds(b * N + start, 400)],
                            spmem_acc.at[pl.ds(start, 400)])
            def _zt(i, _):
                msg_v[0, pl.ds(i * 16, 16)] = jnp.zeros((16,), jnp.float32)
                return 0
            lax.fori_loop(0, D // 16, _zt, 0)
            for tr in range(16):
                pltpu.sync_copy(msg_v.at[0], spmem_acc.at[N + tr])

        plsc.subcore_barrier()

        def _msg_chunk(i, _):
            cid = t + i * NT
            @pl.when(cid < NCHUNKS)
            def _():
                e0 = ebase + cid * CHUNK_E
                pltpu.sync_copy(srcrep_hbm.at[pl.ds(e0, CHUNK_E)], idx_v)
                pltpu.async_copy(g_hbm.at[idx_v], msg_v, sem).wait()
                pltpu.sync_copy(dst_hbm.at[pl.ds(e0, CHUNK_E)], idx_v)
                pltpu.sync_copy(msg_v, spmem_acc.at[idx_v], add=True)
            return 0
        lax.fori_loop(0, (NCHUNKS + NT - 1) // NT, _msg_chunk, 0)

        plsc.subcore_barrier()

        @pl.when(~last)
        def _():
            pltpu.sync_copy(spmem_acc.at[pl.ds(start, NODES_T)],
                            acc_hbm.at[pl.ds(b * N + start, NODES_T)])

        @pl.when(last)
        def _():
            pltpu.sync_copy(spmem_acc.at[pl.ds(start, 400)],
                            acc_hbm.at[pl.ds(b * N + start, 400)])

        plsc.subcore_barrier()


def _stats_body(acc_ref, dinv_ref, bias_ref, stats_ref, accum):
    step = pl.program_id(0)

    @pl.when(step == 0)
    def _():
        accum[0] = 0.0
        accum[1] = 0.0

    y = dinv_ref[...] * acc_ref[...] + bias_ref[...]
    accum[0] += jnp.sum(y)
    accum[1] += jnp.sum(y * y)

    @pl.when(step == pl.num_programs(0) - 1)
    def _():
        stats_ref[0] = accum[0]
        stats_ref[1] = accum[1]


def _final_body(acc_ref, dinv_ref, bias_ref, stats_ref,
                gamma_ref, beta_ref, a_ref, out_ref):
    mu = stats_ref[0] / (NN * D)
    var = stats_ref[1] / (NN * D) - mu * mu
    rs = lax.rsqrt(var + 1e-5)
    y = dinv_ref[...] * acc_ref[...] + bias_ref[...]
    y = (y - mu) * rs * gamma_ref[...] + beta_ref[...]
    out_ref[...] = jnp.where(y >= 0, y, a_ref[...] * y)


def kernel(x, edge_index, edge_mask, W, b, gamma, beta, prelu_a):
    x2 = x[:, 0, :]
    ei32 = edge_index.astype(jnp.int32)
    em32 = edge_mask.astype(jnp.int32)

    h = pl.pallas_call(
        _matmul_body,
        grid=(20,),
        in_specs=[pl.BlockSpec((2000, D), lambda i: (i, 0)),
                  pl.BlockSpec((D, D), lambda i: (0, 0))],
        out_specs=pl.BlockSpec((2000, D), lambda i: (i, 0)),
        out_shape=jax.ShapeDtypeStruct((NN, D), jnp.float32),
    )(x2, W)

    dst_pad, src_rep = pl.pallas_call(
        _edge_prep_body,
        grid=(10,),
        in_specs=[pl.BlockSpec((B, 1000, K), lambda i: (0, i, 0)),
                  pl.BlockSpec((B, 1000, K), lambda i: (0, i, 0))],
        out_specs=[pl.BlockSpec((B, 1000, 16), lambda i: (0, i, 0)),
                   pl.BlockSpec((B, 1000, 16), lambda i: (0, i, 0))],
        out_shape=[jax.ShapeDtypeStruct((B, N, 16), jnp.int32),
                   jax.ShapeDtypeStruct((B, N, 16), jnp.int32)],
    )(ei32, em32)
    dst_flat = dst_pad.reshape(-1)
    src_flat = src_rep.reshape(-1)

    mesh = plsc.VectorSubcoreMesh(core_axis_name="c", subcore_axis_name="s")
    acc, dinv, g = pl.kernel(
        _sc_body,
        out_type=(jax.ShapeDtypeStruct((NN, D), jnp.float32),
                  jax.ShapeDtypeStruct((NN,), jnp.float32),
                  jax.ShapeDtypeStruct((NN, D), jnp.float32)),
        mesh=mesh,
        scratch_types=[
            pltpu.VMEM_SHARED((ACC_ROWS, D), jnp.float32),
            pltpu.VMEM_SHARED((NT * ACC_ROWS,), jnp.float32),
            pltpu.VMEM((CHUNK_E,), jnp.float32),
            pltpu.VMEM((NODES_T,), jnp.float32),
            pltpu.VMEM((CHUNK_E,), jnp.int32),
            pltpu.VMEM((CHUNK_E, D), jnp.float32),
            pltpu.VMEM((GROWS, D), jnp.float32),
            pltpu.VMEM((NODES_T,), jnp.float32),
            pltpu.VMEM((NT * NODES_T,), jnp.float32),
            pltpu.SemaphoreType.DMA,
        ],
    )(h, dst_flat, src_flat)

    dinv2 = dinv.reshape(NN, 1)
    bias2 = b.reshape(1, D)
    stats = pl.pallas_call(
        _stats_body,
        grid=(20,),
        in_specs=[pl.BlockSpec((2000, D), lambda i: (i, 0)),
                  pl.BlockSpec((2000, 1), lambda i: (i, 0)),
                  pl.BlockSpec((1, D), lambda i: (0, 0))],
        out_specs=pl.BlockSpec(memory_space=pltpu.SMEM),
        out_shape=jax.ShapeDtypeStruct((2,), jnp.float32),
        scratch_shapes=[pltpu.SMEM((2,), jnp.float32)],
    )(acc, dinv2, bias2)

    out = pl.pallas_call(
        _final_body,
        grid=(20,),
        in_specs=[pl.BlockSpec((2000, D), lambda i: (i, 0)),
                  pl.BlockSpec((2000, 1), lambda i: (i, 0)),
                  pl.BlockSpec((1, D), lambda i: (0, 0)),
                  pl.BlockSpec(memory_space=pltpu.SMEM),
                  pl.BlockSpec((1, D), lambda i: (0, 0)),
                  pl.BlockSpec((1, D), lambda i: (0, 0)),
                  pl.BlockSpec((1, 1), lambda i: (0, 0))],
        out_specs=pl.BlockSpec((2000, D), lambda i: (i, 0)),
        out_shape=jax.ShapeDtypeStruct((NN, D), jnp.float32),
    )(acc, dinv2, bias2, stats, gamma.reshape(1, D), beta.reshape(1, D),
      prelu_a.reshape(1, 1))
    return out


# speedup vs baseline: 22.9758x; 1.5685x over previous
import functools

import jax
import jax.numpy as jnp
from jax import lax
from jax.experimental import pallas as pl
from jax.experimental.pallas import tpu as pltpu
from jax.experimental.pallas import tpu_sc as plsc

B = 4
N = 10000
K = 17
D = 128
NN = B * N
EB = N * 16
TRASH = N
ACC_ROWS = N + 16
NT = 16
NODES_T = 640
CHUNK_E = 128
NCHUNKS = EB // CHUNK_E
GROWS = 80
NSUP = (NCHUNKS + 7) // 8
NSUP_T = (NSUP + NT - 1) // NT


def _matmul_body(x_ref, w_ref, h_ref):
    h_ref[...] = jnp.dot(x_ref[...], w_ref[...],
                         preferred_element_type=jnp.float32)


def _edge_prep_body(ei_ref, em_ref, dst_ref, src_ref):
    step = pl.program_id(0)
    ei = ei_ref[...][:, :, 1:]
    em = em_ref[...][:, :, 1:]
    dst_ref[...] = jnp.where(em != 0, ei, TRASH)
    bidx = lax.broadcasted_iota(jnp.int32, (B, 1000, 16), 0)
    iidx = lax.broadcasted_iota(jnp.int32, (B, 1000, 16), 1) + step * 1000
    src_ref[...] = bidx * N + iidx


def _sc_body(h_hbm, dst_hbm, srcrep_hbm,
             acc_hbm, dinv_hbm, g_hbm,
             spmem_acc, spmem_degf,
             one_v, zer_v, sidx_v, didx_v, msg_v, dnv_v, red_v,
             sem_g, sem_d, sem_r):
    c = lax.axis_index("c")
    t = lax.axis_index("s")
    last = t == NT - 1
    start = t * NODES_T
    dbase = t * ACC_ROWS

    def _fill_one(i, _):
        one_v[pl.ds(i * 16, 16)] = jnp.full((16,), 1.0, jnp.float32)
        return 0
    lax.fori_loop(0, CHUNK_E // 16, _fill_one, 0)

    def _fill_zer(i, _):
        zer_v[pl.ds(i * 16, 16)] = jnp.zeros((16,), jnp.float32)
        return 0
    lax.fori_loop(0, NODES_T // 16, _fill_zer, 0)

    for lb in range(2):
        b = c * 2 + lb
        ebase = b * EB

        for j in range(15):
            pltpu.async_copy(zer_v,
                             spmem_degf.at[pl.ds(dbase + j * NODES_T,
                                                 NODES_T)], sem_r)
        pltpu.async_copy(zer_v.at[pl.ds(0, ACC_ROWS - 15 * NODES_T)],
                         spmem_degf.at[pl.ds(dbase + 15 * NODES_T,
                                             ACC_ROWS - 15 * NODES_T)],
                         sem_r)
        for j in range(15):
            pltpu.make_async_copy(
                zer_v, spmem_degf.at[pl.ds(dbase + j * NODES_T,
                                           NODES_T)], sem_r).wait()
        pltpu.make_async_copy(
            zer_v.at[pl.ds(0, ACC_ROWS - 15 * NODES_T)],
            spmem_degf.at[pl.ds(dbase + 15 * NODES_T,
                                ACC_ROWS - 15 * NODES_T)], sem_r).wait()

        def _deg_super(u, _):
            pu = lax.rem(u, 2)
            sb = t + u * NT

            @pl.when(u >= 1)
            def _():
                sbp = t + (u - 1) * NT
                for j in range(8):
                    o = (1 - pu) * 1024 + j * CHUNK_E
                    @pl.when(sbp * 8 + j < NCHUNKS)
                    def _():
                        pltpu.make_async_copy(
                            one_v,
                            spmem_degf.at[didx_v.at[pl.ds(o, CHUNK_E)]],
                            sem_d).wait()

            @pl.when(sb < NSUP - 1)
            def _():
                pltpu.sync_copy(dst_hbm.at[pl.ds(ebase + sb * 1024, 1024)],
                                didx_v.at[pl.ds(pu * 1024, 1024)])

            @pl.when(sb == NSUP - 1)
            def _():
                pltpu.sync_copy(dst_hbm.at[pl.ds(ebase + sb * 1024, 256)],
                                didx_v.at[pl.ds(pu * 1024, 256)])

            for j in range(8):
                o = pu * 1024 + j * CHUNK_E
                @pl.when(sb * 8 + j < NCHUNKS)
                def _():
                    for l in range(CHUNK_E // 16):
                        didx_v[pl.ds(o + l * 16, 16)] = (
                            didx_v[pl.ds(o + l * 16, 16)] + dbase)
                    pltpu.async_copy(
                        one_v,
                        spmem_degf.at[didx_v.at[pl.ds(o, CHUNK_E)]],
                        sem_d, add=True)
            return 0
        lax.fori_loop(0, NSUP_T, _deg_super, 0)
        for j in range(8):
            o = ((NSUP_T - 1) % 2) * 1024 + j * CHUNK_E
            sbl = t + (NSUP_T - 1) * NT
            @pl.when(sbl * 8 + j < NCHUNKS)
            def _():
                pltpu.make_async_copy(
                    one_v, spmem_degf.at[didx_v.at[pl.ds(o, CHUNK_E)]],
                    sem_d).wait()

        plsc.subcore_barrier()

        cnt_nodes = jnp.where(last, 400, NODES_T)
        for m in range(NODES_T // 128):
            node0 = start + m * 128

            def _fire(sz):
                for k in range(NT):
                    pltpu.async_copy(
                        spmem_degf.at[pl.ds(k * ACC_ROWS + node0, sz)],
                        red_v.at[pl.ds(k * 128, sz)], sem_r)

            def _drain(sz):
                for k in range(NT):
                    pltpu.make_async_copy(
                        spmem_degf.at[pl.ds(k * ACC_ROWS + node0, sz)],
                        red_v.at[pl.ds(k * 128, sz)], sem_r).wait()

            if m < 3:
                _fire(128); _drain(128)
            elif m == 3:
                @pl.when(~last)
                def _():
                    _fire(128); _drain(128)

                @pl.when(last)
                def _():
                    _fire(32); _drain(32)
            else:
                @pl.when(~last)
                def _():
                    _fire(128); _drain(128)

            def _newton(i, _):
                @pl.when(m * 128 + i * 16 < cnt_nodes)
                def _():
                    x = jnp.full((16,), 1.0, jnp.float32)
                    for k in range(NT):
                        x = x + red_v[pl.ds(k * 128 + i * 16, 16)]
                    bits = lax.bitcast_convert_type(x, jnp.int32)
                    seed = jnp.full((16,), 0x5F3759DF, jnp.int32) - (
                        lax.shift_right_logical(bits, 1))
                    y = lax.bitcast_convert_type(seed, jnp.float32)
                    for _it in range(4):
                        y = y * (1.5 - 0.5 * x * y * y)
                    dnv_v[pl.ds(m * 128 + i * 16, 16)] = y
                return 0
            lax.fori_loop(0, 8, _newton, 0)

        plsc.subcore_barrier()

        @pl.when(~last)
        def _():
            pltpu.sync_copy(dnv_v.at[pl.ds(0, NODES_T)],
                            dinv_hbm.at[pl.ds(b * N + start, NODES_T)])

        @pl.when(last)
        def _():
            pltpu.sync_copy(dnv_v.at[pl.ds(0, 400)],
                            dinv_hbm.at[pl.ds(b * N + start, 400)])

        def _g_chunk(ci, _):
            r0 = ci * GROWS
            @pl.when(r0 < cnt_nodes)
            def _():
                gbase = b * N + start + r0
                pltpu.sync_copy(h_hbm.at[pl.ds(gbase, GROWS)],
                                msg_v.at[0, pl.ds(0, GROWS)])

                def _scale16(rr, _2):
                    dvec = dnv_v[pl.ds(r0 + rr * 16, 16)]
                    for l in range(16):
                        dv = jnp.full((16,), 1.0, jnp.float32) * dvec[l]
                        r = rr * 16 + l
                        for j in range(D // 16):
                            msg_v[0, r, pl.ds(j * 16, 16)] = (
                                msg_v[0, r, pl.ds(j * 16, 16)] * dv)
                    return 0
                lax.fori_loop(0, GROWS // 16, _scale16, 0)
                pltpu.sync_copy(msg_v.at[0, pl.ds(0, GROWS)],
                                g_hbm.at[pl.ds(gbase, GROWS)])
            return 0
        lax.fori_loop(0, NODES_T // GROWS, _g_chunk, 0)

    plsc.subcore_barrier()

    for lb in range(2):
        b = c * 2 + lb
        ebase = b * EB

        @pl.when(~last)
        def _():
            pltpu.sync_copy(g_hbm.at[pl.ds(b * N + start, NODES_T)],
                            spmem_acc.at[pl.ds(start, NODES_T)])

        @pl.when(last)
        def _():
            pltpu.sync_copy(g_hbm.at[pl.ds(b * N + start, 400)],
                            spmem_acc.at[pl.ds(start, 400)])
            def _zt(i, _):
                msg_v[0, 0, pl.ds(i * 16, 16)] = jnp.zeros((16,),
                                                           jnp.float32)
                return 0
            lax.fori_loop(0, D // 16, _zt, 0)
            for tr in range(16):
                pltpu.sync_copy(msg_v.at[0, 0], spmem_acc.at[N + tr])

        plsc.subcore_barrier()

        def _msg_super(u, _):
            pu = lax.rem(u, 2)
            sb = t + u * NT

            @pl.when(sb < NSUP - 1)
            def _():
                pltpu.sync_copy(dst_hbm.at[pl.ds(ebase + sb * 1024, 1024)],
                                didx_v.at[pl.ds(pu * 1024, 1024)])
                pltpu.sync_copy(
                    srcrep_hbm.at[pl.ds(ebase + sb * 1024, 1024)],
                    sidx_v.at[pl.ds(pu * 1024, 1024)])

            @pl.when(sb == NSUP - 1)
            def _():
                pltpu.sync_copy(dst_hbm.at[pl.ds(ebase + sb * 1024, 256)],
                                didx_v.at[pl.ds(pu * 1024, 256)])
                pltpu.sync_copy(
                    srcrep_hbm.at[pl.ds(ebase + sb * 1024, 256)],
                    sidx_v.at[pl.ds(pu * 1024, 256)])

            @pl.when(sb * 8 < NCHUNKS)
            def _():
                pltpu.async_copy(
                    g_hbm.at[sidx_v.at[pl.ds(pu * 1024, CHUNK_E)]],
                    msg_v.at[0], sem_g)

            for j in range(8):
                o = pu * 1024 + j * CHUNK_E
                p = j % 2
                if j < 7:
                    @pl.when(sb * 8 + j + 1 < NCHUNKS)
                    def _():
                        pltpu.async_copy(
                            g_hbm.at[sidx_v.at[pl.ds(o + CHUNK_E,
                                                     CHUNK_E)]],
                            msg_v.at[1 - p], sem_g)

                @pl.when(sb * 8 + j < NCHUNKS)
                def _():
                    pltpu.make_async_copy(
                        g_hbm.at[sidx_v.at[pl.ds(o, CHUNK_E)]],
                        msg_v.at[p], sem_g).wait()
                    pltpu.sync_copy(
                        msg_v.at[p],
                        spmem_acc.at[didx_v.at[pl.ds(o, CHUNK_E)]],
                        add=True)
            return 0
        lax.fori_loop(0, NSUP_T, _msg_super, 0)

        plsc.subcore_barrier()

        @pl.when(~last)
        def _():
            pltpu.sync_copy(spmem_acc.at[pl.ds(start, NODES_T)],
                            acc_hbm.at[pl.ds(b * N + start, NODES_T)])

        @pl.when(last)
        def _():
            pltpu.sync_copy(spmem_acc.at[pl.ds(start, 400)],
                            acc_hbm.at[pl.ds(b * N + start, 400)])

        plsc.subcore_barrier()


def _stats_body(acc_ref, dinv_ref, bias_ref, stats_ref, accum):
    step = pl.program_id(0)

    @pl.when(step == 0)
    def _():
        accum[0] = 0.0
        accum[1] = 0.0

    y = dinv_ref[...] * acc_ref[...] + bias_ref[...]
    accum[0] += jnp.sum(y)
    accum[1] += jnp.sum(y * y)

    @pl.when(step == pl.num_programs(0) - 1)
    def _():
        stats_ref[0] = accum[0]
        stats_ref[1] = accum[1]


def _final_body(acc_ref, dinv_ref, bias_ref, stats_ref,
                gamma_ref, beta_ref, a_ref, out_ref):
    mu = stats_ref[0] / (NN * D)
    var = stats_ref[1] / (NN * D) - mu * mu
    rs = lax.rsqrt(var + 1e-5)
    y = dinv_ref[...] * acc_ref[...] + bias_ref[...]
    y = (y - mu) * rs * gamma_ref[...] + beta_ref[...]
    out_ref[...] = jnp.where(y >= 0, y, a_ref[...] * y)


def kernel(x, edge_index, edge_mask, W, b, gamma, beta, prelu_a):
    x2 = x[:, 0, :]
    ei32 = edge_index.astype(jnp.int32)
    em32 = edge_mask.astype(jnp.int32)

    h = pl.pallas_call(
        _matmul_body,
        grid=(20,),
        in_specs=[pl.BlockSpec((2000, D), lambda i: (i, 0)),
                  pl.BlockSpec((D, D), lambda i: (0, 0))],
        out_specs=pl.BlockSpec((2000, D), lambda i: (i, 0)),
        out_shape=jax.ShapeDtypeStruct((NN, D), jnp.float32),
    )(x2, W)

    dst_pad, src_rep = pl.pallas_call(
        _edge_prep_body,
        grid=(10,),
        in_specs=[pl.BlockSpec((B, 1000, K), lambda i: (0, i, 0)),
                  pl.BlockSpec((B, 1000, K), lambda i: (0, i, 0))],
        out_specs=[pl.BlockSpec((B, 1000, 16), lambda i: (0, i, 0)),
                   pl.BlockSpec((B, 1000, 16), lambda i: (0, i, 0))],
        out_shape=[jax.ShapeDtypeStruct((B, N, 16), jnp.int32),
                   jax.ShapeDtypeStruct((B, N, 16), jnp.int32)],
    )(ei32, em32)
    dst_flat = dst_pad.reshape(-1)
    src_flat = src_rep.reshape(-1)

    mesh = plsc.VectorSubcoreMesh(core_axis_name="c", subcore_axis_name="s")
    acc, dinv, g = pl.kernel(
        _sc_body,
        out_type=(jax.ShapeDtypeStruct((NN, D), jnp.float32),
                  jax.ShapeDtypeStruct((NN,), jnp.float32),
                  jax.ShapeDtypeStruct((NN, D), jnp.float32)),
        mesh=mesh,
        scratch_types=[
            pltpu.VMEM_SHARED((ACC_ROWS, D), jnp.float32),
            pltpu.VMEM_SHARED((NT * ACC_ROWS,), jnp.float32),
            pltpu.VMEM((CHUNK_E,), jnp.float32),
            pltpu.VMEM((NODES_T,), jnp.float32),
            pltpu.VMEM((2048,), jnp.int32),
            pltpu.VMEM((2048,), jnp.int32),
            pltpu.VMEM((2, CHUNK_E, D), jnp.float32),
            pltpu.VMEM((NODES_T,), jnp.float32),
            pltpu.VMEM((NT * 128,), jnp.float32),
            pltpu.SemaphoreType.DMA,
            pltpu.SemaphoreType.DMA,
            pltpu.SemaphoreType.DMA,
        ],
    )(h, dst_flat, src_flat)

    dinv2 = dinv.reshape(NN, 1)
    bias2 = b.reshape(1, D)
    stats = pl.pallas_call(
        _stats_body,
        grid=(20,),
        in_specs=[pl.BlockSpec((2000, D), lambda i: (i, 0)),
                  pl.BlockSpec((2000, 1), lambda i: (i, 0)),
                  pl.BlockSpec((1, D), lambda i: (0, 0))],
        out_specs=pl.BlockSpec(memory_space=pltpu.SMEM),
        out_shape=jax.ShapeDtypeStruct((2,), jnp.float32),
        scratch_shapes=[pltpu.SMEM((2,), jnp.float32)],
    )(acc, dinv2, bias2)

    out = pl.pallas_call(
        _final_body,
        grid=(20,),
        in_specs=[pl.BlockSpec((2000, D), lambda i: (i, 0)),
                  pl.BlockSpec((2000, 1), lambda i: (i, 0)),
                  pl.BlockSpec((1, D), lambda i: (0, 0)),
                  pl.BlockSpec(memory_space=pltpu.SMEM),
                  pl.BlockSpec((1, D), lambda i: (0, 0)),
                  pl.BlockSpec((1, D), lambda i: (0, 0)),
                  pl.BlockSpec((1, 1), lambda i: (0, 0))],
        out_specs=pl.BlockSpec((2000, D), lambda i: (i, 0)),
        out_shape=jax.ShapeDtypeStruct((NN, D), jnp.float32),
    )(acc, dinv2, bias2, stats, gamma.reshape(1, D), beta.reshape(1, D),
      prelu_a.reshape(1, 1))
    return out


# speedup vs baseline: 23.8870x; 1.0397x over previous
import functools

import jax
import jax.numpy as jnp
from jax import lax
from jax.experimental import pallas as pl
from jax.experimental.pallas import tpu as pltpu
from jax.experimental.pallas import tpu_sc as plsc

B = 4
N = 10000
K = 17
D = 128
NN = B * N
EB = N * 16
TRASH = N
ACC_ROWS = N + 16
NT = 16
NODES_T = 640
CHUNK_E = 128
NCHUNKS = EB // CHUNK_E
GROWS = 80
NSUP = (NCHUNKS + 7) // 8
NSUP_T = (NSUP + NT - 1) // NT


def _matmul_body(x_ref, w_ref, h_ref):
    h_ref[...] = jnp.dot(x_ref[...], w_ref[...],
                         preferred_element_type=jnp.float32)


def _edge_prep_body(ei_ref, em_ref, dst_ref):
    ei = ei_ref[...][:, :, 1:]
    em = em_ref[...][:, :, 1:]
    dst_ref[...] = jnp.where(em != 0, ei, TRASH)


def _sc_body(h_hbm, dst_hbm,
             acc_hbm, dinv_hbm, g_hbm,
             spmem_acc, spmem_degf,
             one_v, zer_v, sidx_v, didx_v, msg_v, dnv_v, red_v,
             sem_g, sem_d, sem_r):
    c = lax.axis_index("c")
    t = lax.axis_index("s")
    last = t == NT - 1
    start = t * NODES_T
    dbase = t * ACC_ROWS

    def _fill_one(i, _):
        one_v[pl.ds(i * 16, 16)] = jnp.full((16,), 1.0, jnp.float32)
        return 0
    lax.fori_loop(0, CHUNK_E // 16, _fill_one, 0)

    def _fill_zer(i, _):
        zer_v[pl.ds(i * 16, 16)] = jnp.zeros((16,), jnp.float32)
        return 0
    lax.fori_loop(0, NODES_T // 16, _fill_zer, 0)

    for lb in range(2):
        b = c * 2 + lb
        ebase = b * EB

        for j in range(15):
            pltpu.async_copy(zer_v,
                             spmem_degf.at[pl.ds(dbase + j * NODES_T,
                                                 NODES_T)], sem_r)
        pltpu.async_copy(zer_v.at[pl.ds(0, ACC_ROWS - 15 * NODES_T)],
                         spmem_degf.at[pl.ds(dbase + 15 * NODES_T,
                                             ACC_ROWS - 15 * NODES_T)],
                         sem_r)
        for j in range(15):
            pltpu.make_async_copy(
                zer_v, spmem_degf.at[pl.ds(dbase + j * NODES_T,
                                           NODES_T)], sem_r).wait()
        pltpu.make_async_copy(
            zer_v.at[pl.ds(0, ACC_ROWS - 15 * NODES_T)],
            spmem_degf.at[pl.ds(dbase + 15 * NODES_T,
                                ACC_ROWS - 15 * NODES_T)], sem_r).wait()

        def _deg_super(u, _):
            pu = lax.rem(u, 2)
            sb = t + u * NT

            @pl.when(u >= 1)
            def _():
                sbp = t + (u - 1) * NT
                for j in range(8):
                    o = (1 - pu) * 1024 + j * CHUNK_E
                    @pl.when(sbp * 8 + j < NCHUNKS)
                    def _():
                        pltpu.make_async_copy(
                            one_v,
                            spmem_degf.at[didx_v.at[pl.ds(o, CHUNK_E)]],
                            sem_d).wait()

            @pl.when(sb < NSUP - 1)
            def _():
                pltpu.sync_copy(dst_hbm.at[pl.ds(ebase + sb * 1024, 1024)],
                                didx_v.at[pl.ds(pu * 1024, 1024)])

            @pl.when(sb == NSUP - 1)
            def _():
                pltpu.sync_copy(dst_hbm.at[pl.ds(ebase + sb * 1024, 256)],
                                didx_v.at[pl.ds(pu * 1024, 256)])

            for j in range(8):
                o = pu * 1024 + j * CHUNK_E
                @pl.when(sb * 8 + j < NCHUNKS)
                def _():
                    for l in range(CHUNK_E // 16):
                        didx_v[pl.ds(o + l * 16, 16)] = (
                            didx_v[pl.ds(o + l * 16, 16)] + dbase)
                    pltpu.async_copy(
                        one_v,
                        spmem_degf.at[didx_v.at[pl.ds(o, CHUNK_E)]],
                        sem_d, add=True)
            return 0
        lax.fori_loop(0, NSUP_T, _deg_super, 0)
        for j in range(8):
            o = ((NSUP_T - 1) % 2) * 1024 + j * CHUNK_E
            sbl = t + (NSUP_T - 1) * NT
            @pl.when(sbl * 8 + j < NCHUNKS)
            def _():
                pltpu.make_async_copy(
                    one_v, spmem_degf.at[didx_v.at[pl.ds(o, CHUNK_E)]],
                    sem_d).wait()

        plsc.subcore_barrier()

        cnt_nodes = jnp.where(last, 400, NODES_T)
        for m in range(NODES_T // 128):
            node0 = start + m * 128

            def _fire(sz):
                for k in range(NT):
                    pltpu.async_copy(
                        spmem_degf.at[pl.ds(k * ACC_ROWS + node0, sz)],
                        red_v.at[pl.ds(k * 128, sz)], sem_r)

            def _drain(sz):
                for k in range(NT):
                    pltpu.make_async_copy(
                        spmem_degf.at[pl.ds(k * ACC_ROWS + node0, sz)],
                        red_v.at[pl.ds(k * 128, sz)], sem_r).wait()

            if m < 3:
                _fire(128); _drain(128)
            elif m == 3:
                @pl.when(~last)
                def _():
                    _fire(128); _drain(128)

                @pl.when(last)
                def _():
                    _fire(32); _drain(32)
            else:
                @pl.when(~last)
                def _():
                    _fire(128); _drain(128)

            def _newton(i, _):
                @pl.when(m * 128 + i * 16 < cnt_nodes)
                def _():
                    x = jnp.full((16,), 1.0, jnp.float32)
                    for k in range(NT):
                        x = x + red_v[pl.ds(k * 128 + i * 16, 16)]
                    bits = lax.bitcast_convert_type(x, jnp.int32)
                    seed = jnp.full((16,), 0x5F3759DF, jnp.int32) - (
                        lax.shift_right_logical(bits, 1))
                    y = lax.bitcast_convert_type(seed, jnp.float32)
                    for _it in range(4):
                        y = y * (1.5 - 0.5 * x * y * y)
                    dnv_v[pl.ds(m * 128 + i * 16, 16)] = y
                return 0
            lax.fori_loop(0, 8, _newton, 0)

        plsc.subcore_barrier()

        @pl.when(~last)
        def _():
            pltpu.sync_copy(dnv_v.at[pl.ds(0, NODES_T)],
                            dinv_hbm.at[pl.ds(b * N + start, NODES_T)])

        @pl.when(last)
        def _():
            pltpu.sync_copy(dnv_v.at[pl.ds(0, 400)],
                            dinv_hbm.at[pl.ds(b * N + start, 400)])

        def _g_chunk(ci, _):
            r0 = ci * GROWS
            @pl.when(r0 < cnt_nodes)
            def _():
                gbase = b * N + start + r0
                pltpu.sync_copy(h_hbm.at[pl.ds(gbase, GROWS)],
                                msg_v.at[0, pl.ds(0, GROWS)])

                def _scale16(rr, _2):
                    dvec = dnv_v[pl.ds(r0 + rr * 16, 16)]
                    for l in range(16):
                        dv = jnp.full((16,), 1.0, jnp.float32) * dvec[l]
                        r = rr * 16 + l
                        for j in range(D // 16):
                            msg_v[0, r, pl.ds(j * 16, 16)] = (
                                msg_v[0, r, pl.ds(j * 16, 16)] * dv)
                    return 0
                lax.fori_loop(0, GROWS // 16, _scale16, 0)
                pltpu.sync_copy(msg_v.at[0, pl.ds(0, GROWS)],
                                g_hbm.at[pl.ds(gbase, GROWS)])
            return 0
        lax.fori_loop(0, NODES_T // GROWS, _g_chunk, 0)

    plsc.subcore_barrier()

    for lb in range(2):
        b = c * 2 + lb
        ebase = b * EB

        @pl.when(~last)
        def _():
            pltpu.sync_copy(g_hbm.at[pl.ds(b * N + start, NODES_T)],
                            spmem_acc.at[pl.ds(start, NODES_T)])

        @pl.when(last)
        def _():
            pltpu.sync_copy(g_hbm.at[pl.ds(b * N + start, 400)],
                            spmem_acc.at[pl.ds(start, 400)])
            def _zt(i, _):
                msg_v[0, 0, pl.ds(i * 16, 16)] = jnp.zeros((16,),
                                                           jnp.float32)
                return 0
            lax.fori_loop(0, D // 16, _zt, 0)
            for tr in range(16):
                pltpu.sync_copy(msg_v.at[0, 0], spmem_acc.at[N + tr])

        plsc.subcore_barrier()

        def _msg_super(u, _):
            pu = lax.rem(u, 2)
            sb = t + u * NT

            @pl.when(sb < NSUP - 1)
            def _():
                pltpu.sync_copy(dst_hbm.at[pl.ds(ebase + sb * 1024, 1024)],
                                didx_v.at[pl.ds(pu * 1024, 1024)])

            @pl.when(sb == NSUP - 1)
            def _():
                pltpu.sync_copy(dst_hbm.at[pl.ds(ebase + sb * 1024, 256)],
                                didx_v.at[pl.ds(pu * 1024, 256)])

            for j in range(8):
                o = pu * 1024 + j * CHUNK_E
                @pl.when(sb * 8 + j < NCHUNKS)
                def _():
                    node0 = b * N + (sb * 8 + j) * 8
                    for l in range(8):
                        sidx_v[pl.ds(o + l * 16, 16)] = (
                            jnp.zeros((16,), jnp.int32) + node0 + l)

            @pl.when(sb * 8 < NCHUNKS)
            def _():
                pltpu.async_copy(
                    g_hbm.at[sidx_v.at[pl.ds(pu * 1024, CHUNK_E)]],
                    msg_v.at[0], sem_g)

            for j in range(8):
                o = pu * 1024 + j * CHUNK_E
                p = j % 2
                if j < 7:
                    @pl.when(sb * 8 + j + 1 < NCHUNKS)
                    def _():
                        pltpu.async_copy(
                            g_hbm.at[sidx_v.at[pl.ds(o + CHUNK_E,
                                                     CHUNK_E)]],
                            msg_v.at[1 - p], sem_g)

                @pl.when(sb * 8 + j < NCHUNKS)
                def _():
                    pltpu.make_async_copy(
                        g_hbm.at[sidx_v.at[pl.ds(o, CHUNK_E)]],
                        msg_v.at[p], sem_g).wait()
                    pltpu.sync_copy(
                        msg_v.at[p],
                        spmem_acc.at[didx_v.at[pl.ds(o, CHUNK_E)]],
                        add=True)
            return 0
        lax.fori_loop(0, NSUP_T, _msg_super, 0)

        plsc.subcore_barrier()

        @pl.when(~last)
        def _():
            pltpu.sync_copy(spmem_acc.at[pl.ds(start, NODES_T)],
                            acc_hbm.at[pl.ds(b * N + start, NODES_T)])

        @pl.when(last)
        def _():
            pltpu.sync_copy(spmem_acc.at[pl.ds(start, 400)],
                            acc_hbm.at[pl.ds(b * N + start, 400)])

        plsc.subcore_barrier()


def _ln_body(acc_ref, dinv_ref, bias_ref, gamma_ref, beta_ref, a_ref,
             out_ref, accum):
    p = pl.program_id(0)
    i = pl.program_id(1)
    y = dinv_ref[...] * acc_ref[...] + bias_ref[...]

    @pl.when((p == 0) & (i == 0))
    def _():
        accum[0] = 0.0
        accum[1] = 0.0

    @pl.when(p == 0)
    def _():
        accum[0] += jnp.sum(y)
        accum[1] += jnp.sum(y * y)

    @pl.when(p == 1)
    def _():
        mu = accum[0] / (NN * D)
        var = accum[1] / (NN * D) - mu * mu
        rs = lax.rsqrt(var + 1e-5)
        z = (y - mu) * rs * gamma_ref[...] + beta_ref[...]
        out_ref[...] = jnp.where(z >= 0, z, a_ref[...] * z)


def kernel(x, edge_index, edge_mask, W, b, gamma, beta, prelu_a):
    x2 = x[:, 0, :]
    ei32 = edge_index.astype(jnp.int32)
    em32 = edge_mask.astype(jnp.int32)

    h = pl.pallas_call(
        _matmul_body,
        grid=(20,),
        in_specs=[pl.BlockSpec((2000, D), lambda i: (i, 0)),
                  pl.BlockSpec((D, D), lambda i: (0, 0))],
        out_specs=pl.BlockSpec((2000, D), lambda i: (i, 0)),
        out_shape=jax.ShapeDtypeStruct((NN, D), jnp.float32),
    )(x2, W)

    dst_pad = pl.pallas_call(
        _edge_prep_body,
        grid=(10,),
        in_specs=[pl.BlockSpec((B, 1000, K), lambda i: (0, i, 0)),
                  pl.BlockSpec((B, 1000, K), lambda i: (0, i, 0))],
        out_specs=pl.BlockSpec((B, 1000, 16), lambda i: (0, i, 0)),
        out_shape=jax.ShapeDtypeStruct((B, N, 16), jnp.int32),
    )(ei32, em32)
    dst_flat = dst_pad.reshape(-1)

    mesh = plsc.VectorSubcoreMesh(core_axis_name="c", subcore_axis_name="s")
    acc, dinv, g = pl.kernel(
        _sc_body,
        out_type=(jax.ShapeDtypeStruct((NN, D), jnp.float32),
                  jax.ShapeDtypeStruct((NN,), jnp.float32),
                  jax.ShapeDtypeStruct((NN, D), jnp.float32)),
        mesh=mesh,
        scratch_types=[
            pltpu.VMEM_SHARED((ACC_ROWS, D), jnp.float32),
            pltpu.VMEM_SHARED((NT * ACC_ROWS,), jnp.float32),
            pltpu.VMEM((CHUNK_E,), jnp.float32),
            pltpu.VMEM((NODES_T,), jnp.float32),
            pltpu.VMEM((2048,), jnp.int32),
            pltpu.VMEM((2048,), jnp.int32),
            pltpu.VMEM((2, CHUNK_E, D), jnp.float32),
            pltpu.VMEM((NODES_T,), jnp.float32),
            pltpu.VMEM((NT * 128,), jnp.float32),
            pltpu.SemaphoreType.DMA,
            pltpu.SemaphoreType.DMA,
            pltpu.SemaphoreType.DMA,
        ],
    )(h, dst_flat)

    dinv2 = dinv.reshape(NN, 1)
    bias2 = b.reshape(1, D)
    out = pl.pallas_call(
        _ln_body,
        grid=(2, 20),
        in_specs=[pl.BlockSpec((2000, D), lambda p, i: (i, 0)),
                  pl.BlockSpec((2000, 1), lambda p, i: (i, 0)),
                  pl.BlockSpec((1, D), lambda p, i: (0, 0)),
                  pl.BlockSpec((1, D), lambda p, i: (0, 0)),
                  pl.BlockSpec((1, D), lambda p, i: (0, 0)),
                  pl.BlockSpec((1, 1), lambda p, i: (0, 0))],
        out_specs=pl.BlockSpec((2000, D), lambda p, i: (i, 0)),
        out_shape=jax.ShapeDtypeStruct((NN, D), jnp.float32),
        scratch_shapes=[pltpu.SMEM((2,), jnp.float32)],
    )(acc, dinv2, bias2, gamma.reshape(1, D), beta.reshape(1, D),
      prelu_a.reshape(1, 1))
    return out


# speedup vs baseline: 26.2001x; 1.0968x over previous
import functools

import jax
import jax.numpy as jnp
from jax import lax
from jax.experimental import pallas as pl
from jax.experimental.pallas import tpu as pltpu
from jax.experimental.pallas import tpu_sc as plsc

B = 4
N = 10000
K = 17
D = 128
NN = B * N
EB = N * 16
TRASH = N
ACC_ROWS = N + 16
NT = 16
NODES_T = 640
CHUNK_E = 128
NCHUNKS = EB // CHUNK_E
GROWS = 80
NSUP = (NCHUNKS + 7) // 8
NSUP_T = (NSUP + NT - 1) // NT


def _matmul_body(x_ref, w_ref, h_ref):
    h_ref[...] = jnp.dot(x_ref[...], w_ref[...],
                         preferred_element_type=jnp.float32)


def _edge_prep_body(ei_ref, em_ref, dst_ref):
    ei = ei_ref[...][:, :, 1:]
    em = em_ref[...][:, :, 1:]
    dst_ref[...] = jnp.where(em != 0, ei, TRASH)


def _sc_body(h_hbm, dst_hbm,
             acc_hbm, dinv_hbm, g_hbm,
             spmem_acc, spmem_degf,
             one_v, zer_v, sidx_v, didx_v, msg_v, dnv_v, red_v,
             sem_g, sem_s, sem_d, sem_r):
    c = lax.axis_index("c")
    t = lax.axis_index("s")
    last = t == NT - 1
    start = t * NODES_T
    dbase = t * ACC_ROWS

    def _fill_one(i, _):
        one_v[pl.ds(i * 16, 16)] = jnp.full((16,), 1.0, jnp.float32)
        return 0
    lax.fori_loop(0, CHUNK_E // 16, _fill_one, 0)

    def _fill_zer(i, _):
        zer_v[pl.ds(i * 16, 16)] = jnp.zeros((16,), jnp.float32)
        return 0
    lax.fori_loop(0, NODES_T // 16, _fill_zer, 0)

    for lb in range(2):
        b = c * 2 + lb
        ebase = b * EB

        for j in range(15):
            pltpu.async_copy(zer_v,
                             spmem_degf.at[pl.ds(dbase + j * NODES_T,
                                                 NODES_T)], sem_r)
        pltpu.async_copy(zer_v.at[pl.ds(0, ACC_ROWS - 15 * NODES_T)],
                         spmem_degf.at[pl.ds(dbase + 15 * NODES_T,
                                             ACC_ROWS - 15 * NODES_T)],
                         sem_r)
        for j in range(15):
            pltpu.make_async_copy(
                zer_v, spmem_degf.at[pl.ds(dbase + j * NODES_T,
                                           NODES_T)], sem_r).wait()
        pltpu.make_async_copy(
            zer_v.at[pl.ds(0, ACC_ROWS - 15 * NODES_T)],
            spmem_degf.at[pl.ds(dbase + 15 * NODES_T,
                                ACC_ROWS - 15 * NODES_T)], sem_r).wait()

        def _deg_super(u, _):
            pu = lax.rem(u, 2)
            sb = t + u * NT

            @pl.when(u >= 1)
            def _():
                sbp = t + (u - 1) * NT
                for j in range(8):
                    o = (1 - pu) * 1024 + j * CHUNK_E
                    @pl.when(sbp * 8 + j < NCHUNKS)
                    def _():
                        pltpu.make_async_copy(
                            one_v,
                            spmem_degf.at[didx_v.at[pl.ds(o, CHUNK_E)]],
                            sem_d).wait()

            @pl.when(sb < NSUP - 1)
            def _():
                pltpu.sync_copy(dst_hbm.at[pl.ds(ebase + sb * 1024, 1024)],
                                didx_v.at[pl.ds(pu * 1024, 1024)])

            @pl.when(sb == NSUP - 1)
            def _():
                pltpu.sync_copy(dst_hbm.at[pl.ds(ebase + sb * 1024, 256)],
                                didx_v.at[pl.ds(pu * 1024, 256)])

            for j in range(8):
                o = pu * 1024 + j * CHUNK_E
                @pl.when(sb * 8 + j < NCHUNKS)
                def _():
                    for l in range(CHUNK_E // 16):
                        didx_v[pl.ds(o + l * 16, 16)] = (
                            didx_v[pl.ds(o + l * 16, 16)] + dbase)
                    pltpu.async_copy(
                        one_v,
                        spmem_degf.at[didx_v.at[pl.ds(o, CHUNK_E)]],
                        sem_d, add=True)
            return 0
        lax.fori_loop(0, NSUP_T, _deg_super, 0)
        for j in range(8):
            o = ((NSUP_T - 1) % 2) * 1024 + j * CHUNK_E
            sbl = t + (NSUP_T - 1) * NT
            @pl.when(sbl * 8 + j < NCHUNKS)
            def _():
                pltpu.make_async_copy(
                    one_v, spmem_degf.at[didx_v.at[pl.ds(o, CHUNK_E)]],
                    sem_d).wait()

        plsc.subcore_barrier()

        cnt_nodes = jnp.where(last, 400, NODES_T)
        for m in range(NODES_T // 128):
            node0 = start + m * 128

            def _fire(sz):
                for k in range(NT):
                    pltpu.async_copy(
                        spmem_degf.at[pl.ds(k * ACC_ROWS + node0, sz)],
                        red_v.at[pl.ds(k * 128, sz)], sem_r)

            def _drain(sz):
                for k in range(NT):
                    pltpu.make_async_copy(
                        spmem_degf.at[pl.ds(k * ACC_ROWS + node0, sz)],
                        red_v.at[pl.ds(k * 128, sz)], sem_r).wait()

            if m < 3:
                _fire(128); _drain(128)
            elif m == 3:
                @pl.when(~last)
                def _():
                    _fire(128); _drain(128)

                @pl.when(last)
                def _():
                    _fire(32); _drain(32)
            else:
                @pl.when(~last)
                def _():
                    _fire(128); _drain(128)

            def _newton(i, _):
                @pl.when(m * 128 + i * 16 < cnt_nodes)
                def _():
                    x = jnp.full((16,), 1.0, jnp.float32)
                    for k in range(NT):
                        x = x + red_v[pl.ds(k * 128 + i * 16, 16)]
                    bits = lax.bitcast_convert_type(x, jnp.int32)
                    seed = jnp.full((16,), 0x5F3759DF, jnp.int32) - (
                        lax.shift_right_logical(bits, 1))
                    y = lax.bitcast_convert_type(seed, jnp.float32)
                    for _it in range(4):
                        y = y * (1.5 - 0.5 * x * y * y)
                    dnv_v[pl.ds(m * 128 + i * 16, 16)] = y
                return 0
            lax.fori_loop(0, 8, _newton, 0)

        plsc.subcore_barrier()

        @pl.when(~last)
        def _():
            pltpu.sync_copy(dnv_v.at[pl.ds(0, NODES_T)],
                            dinv_hbm.at[pl.ds(b * N + start, NODES_T)])

        @pl.when(last)
        def _():
            pltpu.sync_copy(dnv_v.at[pl.ds(0, 400)],
                            dinv_hbm.at[pl.ds(b * N + start, 400)])

        def _g_chunk(ci, _):
            r0 = ci * GROWS
            @pl.when(r0 < cnt_nodes)
            def _():
                gbase = b * N + start + r0
                pltpu.sync_copy(h_hbm.at[pl.ds(gbase, GROWS)],
                                msg_v.at[0, pl.ds(0, GROWS)])

                def _scale16(rr, _2):
                    dvec = dnv_v[pl.ds(r0 + rr * 16, 16)]
                    for l in range(16):
                        dv = jnp.full((16,), 1.0, jnp.float32) * dvec[l]
                        r = rr * 16 + l
                        for j in range(D // 16):
                            msg_v[0, r, pl.ds(j * 16, 16)] = (
                                msg_v[0, r, pl.ds(j * 16, 16)] * dv)
                    return 0
                lax.fori_loop(0, GROWS // 16, _scale16, 0)
                pltpu.sync_copy(msg_v.at[0, pl.ds(0, GROWS)],
                                g_hbm.at[pl.ds(gbase, GROWS)])
            return 0
        lax.fori_loop(0, NODES_T // GROWS, _g_chunk, 0)

    plsc.subcore_barrier()

    for lb in range(2):
        b = c * 2 + lb
        ebase = b * EB

        @pl.when(~last)
        def _():
            pltpu.sync_copy(g_hbm.at[pl.ds(b * N + start, NODES_T)],
                            spmem_acc.at[pl.ds(start, NODES_T)])

        @pl.when(last)
        def _():
            pltpu.sync_copy(g_hbm.at[pl.ds(b * N + start, 400)],
                            spmem_acc.at[pl.ds(start, 400)])
            def _zt(i, _):
                msg_v[0, 0, pl.ds(i * 16, 16)] = jnp.zeros((16,),
                                                           jnp.float32)
                return 0
            lax.fori_loop(0, D // 16, _zt, 0)
            for tr in range(16):
                pltpu.sync_copy(msg_v.at[0, 0], spmem_acc.at[N + tr])

        plsc.subcore_barrier()

        def _valid(cc):
            return (t + (cc // 8) * NT) * 8 + lax.rem(cc, 8) < NCHUNKS

        def _off(cc):
            return lax.rem(cc // 8, 2) * 1024 + lax.rem(cc, 8) * CHUNK_E

        def _idx_load(u):
            sb = t + u * NT
            pu = lax.rem(u, 2)

            @pl.when(sb < NSUP - 1)
            def _():
                pltpu.sync_copy(dst_hbm.at[pl.ds(ebase + sb * 1024, 1024)],
                                didx_v.at[pl.ds(pu * 1024, 1024)])

            @pl.when(sb == NSUP - 1)
            def _():
                pltpu.sync_copy(dst_hbm.at[pl.ds(ebase + sb * 1024, 256)],
                                didx_v.at[pl.ds(pu * 1024, 256)])

        def _splat_src(cc):
            node0 = b * N + ((t + (cc // 8) * NT) * 8 + lax.rem(cc, 8)) * 8
            o = _off(cc)
            for l in range(8):
                sidx_v[pl.ds(o + l * 16, 16)] = (
                    jnp.zeros((16,), jnp.int32) + node0 + l)

        def _gather(cc):
            pltpu.async_copy(
                g_hbm.at[sidx_v.at[pl.ds(_off(cc), CHUNK_E)]],
                msg_v.at[lax.rem(cc, 2)], sem_g)

        def _gather_wait(cc):
            pltpu.make_async_copy(
                g_hbm.at[sidx_v.at[pl.ds(_off(cc), CHUNK_E)]],
                msg_v.at[lax.rem(cc, 2)], sem_g).wait()

        def _scat(cc):
            pltpu.async_copy(
                msg_v.at[lax.rem(cc, 2)],
                spmem_acc.at[didx_v.at[pl.ds(_off(cc), CHUNK_E)]],
                sem_s, add=True)

        def _scat_wait(cc):
            pltpu.make_async_copy(
                msg_v.at[lax.rem(cc, 2)],
                spmem_acc.at[didx_v.at[pl.ds(_off(cc), CHUNK_E)]],
                sem_s).wait()

        _idx_load(0)

        @pl.when(_valid(0))
        def _():
            _splat_src(0)
            _gather(0)

        def _slot(cc, _):
            u = cc // 8
            j = lax.rem(cc, 8)

            @pl.when((j == 0) & (u + 1 < NSUP_T))
            def _():
                _idx_load(u + 1)

            @pl.when((cc >= 2) & _valid(cc - 2))
            def _():
                _scat_wait(cc - 2)

            @pl.when((cc + 1 < 8 * NSUP_T) & _valid(cc + 1))
            def _():
                _splat_src(cc + 1)
                _gather(cc + 1)

            @pl.when(_valid(cc))
            def _():
                _gather_wait(cc)
                _scat(cc)
            return 0

        lax.fori_loop(0, 8 * NSUP_T, _slot, 0)
        for dd in range(2):
            cc = 8 * NSUP_T - 2 + dd
            @pl.when(_valid(cc))
            def _():
                _scat_wait(cc)

        plsc.subcore_barrier()

        @pl.when(~last)
        def _():
            pltpu.sync_copy(spmem_acc.at[pl.ds(start, NODES_T)],
                            acc_hbm.at[pl.ds(b * N + start, NODES_T)])

        @pl.when(last)
        def _():
            pltpu.sync_copy(spmem_acc.at[pl.ds(start, 400)],
                            acc_hbm.at[pl.ds(b * N + start, 400)])

        plsc.subcore_barrier()


def _ln_body(acc_ref, dinv_ref, bias_ref, gamma_ref, beta_ref, a_ref,
             out_ref, accum):
    p = pl.program_id(0)
    i = pl.program_id(1)
    y = dinv_ref[...] * acc_ref[...] + bias_ref[...]

    @pl.when((p == 0) & (i == 0))
    def _():
        accum[0] = 0.0
        accum[1] = 0.0

    @pl.when(p == 0)
    def _():
        accum[0] += jnp.sum(y)
        accum[1] += jnp.sum(y * y)

    @pl.when(p == 1)
    def _():
        mu = accum[0] / (NN * D)
        var = accum[1] / (NN * D) - mu * mu
        rs = lax.rsqrt(var + 1e-5)
        z = (y - mu) * rs * gamma_ref[...] + beta_ref[...]
        out_ref[...] = jnp.where(z >= 0, z, a_ref[...] * z)


def kernel(x, edge_index, edge_mask, W, b, gamma, beta, prelu_a):
    x2 = x[:, 0, :]
    ei32 = edge_index.astype(jnp.int32)
    em32 = edge_mask.astype(jnp.int32)

    h = pl.pallas_call(
        _matmul_body,
        grid=(20,),
        in_specs=[pl.BlockSpec((2000, D), lambda i: (i, 0)),
                  pl.BlockSpec((D, D), lambda i: (0, 0))],
        out_specs=pl.BlockSpec((2000, D), lambda i: (i, 0)),
        out_shape=jax.ShapeDtypeStruct((NN, D), jnp.float32),
    )(x2, W)

    dst_pad = pl.pallas_call(
        _edge_prep_body,
        grid=(10,),
        in_specs=[pl.BlockSpec((B, 1000, K), lambda i: (0, i, 0)),
                  pl.BlockSpec((B, 1000, K), lambda i: (0, i, 0))],
        out_specs=pl.BlockSpec((B, 1000, 16), lambda i: (0, i, 0)),
        out_shape=jax.ShapeDtypeStruct((B, N, 16), jnp.int32),
    )(ei32, em32)
    dst_flat = dst_pad.reshape(-1)

    mesh = plsc.VectorSubcoreMesh(core_axis_name="c", subcore_axis_name="s")
    acc, dinv, g = pl.kernel(
        _sc_body,
        out_type=(jax.ShapeDtypeStruct((NN, D), jnp.float32),
                  jax.ShapeDtypeStruct((NN,), jnp.float32),
                  jax.ShapeDtypeStruct((NN, D), jnp.float32)),
        mesh=mesh,
        scratch_types=[
            pltpu.VMEM_SHARED((ACC_ROWS, D), jnp.float32),
            pltpu.VMEM_SHARED((NT * ACC_ROWS,), jnp.float32),
            pltpu.VMEM((CHUNK_E,), jnp.float32),
            pltpu.VMEM((NODES_T,), jnp.float32),
            pltpu.VMEM((2048,), jnp.int32),
            pltpu.VMEM((2048,), jnp.int32),
            pltpu.VMEM((2, CHUNK_E, D), jnp.float32),
            pltpu.VMEM((NODES_T,), jnp.float32),
            pltpu.VMEM((NT * 128,), jnp.float32),
            pltpu.SemaphoreType.DMA,
            pltpu.SemaphoreType.DMA,
            pltpu.SemaphoreType.DMA,
            pltpu.SemaphoreType.DMA,
        ],
    )(h, dst_flat)

    dinv2 = dinv.reshape(NN, 1)
    bias2 = b.reshape(1, D)
    out = pl.pallas_call(
        _ln_body,
        grid=(2, 20),
        in_specs=[pl.BlockSpec((2000, D), lambda p, i: (i, 0)),
                  pl.BlockSpec((2000, 1), lambda p, i: (i, 0)),
                  pl.BlockSpec((1, D), lambda p, i: (0, 0)),
                  pl.BlockSpec((1, D), lambda p, i: (0, 0)),
                  pl.BlockSpec((1, D), lambda p, i: (0, 0)),
                  pl.BlockSpec((1, 1), lambda p, i: (0, 0))],
        out_specs=pl.BlockSpec((2000, D), lambda p, i: (i, 0)),
        out_shape=jax.ShapeDtypeStruct((NN, D), jnp.float32),
        scratch_shapes=[pltpu.SMEM((2,), jnp.float32)],
    )(acc, dinv2, bias2, gamma.reshape(1, D), beta.reshape(1, D),
      prelu_a.reshape(1, 1))
    return out
